# baseline (speedup 1.0000x reference)
import functools

import jax
import jax.numpy as jnp
from jax import lax
from jax.experimental import pallas as pl
from jax.experimental.pallas import tpu as pltpu
from jax.experimental.pallas import tpu_sc as plsc

B = 4096
S = 200
D = 64
NCLS = 10
CH = 100
NCH = S // CH
LANES = 16
DV = D // LANES


def _sc_info():
    try:
        info = plsc.get_sparse_core_info()
        return info.num_cores, info.num_subcores
    except Exception:
        return 2, 16


def _pooled_sum(idx2d, table):
    ncores, nsub = _sc_info()
    nw = ncores * nsub
    bpw = B // nw
    mesh = plsc.VectorSubcoreMesh(
        core_axis_name="c", subcore_axis_name="s",
        num_cores=ncores, num_subcores=nsub)

    @functools.partial(
        pl.kernel,
        out_type=jax.ShapeDtypeStruct((B, D), jnp.float32),
        mesh=mesh,
        compiler_params=pltpu.CompilerParams(use_tc_tiling_on_sc=False),
        scratch_types=[
            pltpu.VMEM((NCH * bpw, CH), jnp.int32),
            pltpu.VMEM((CH, D), jnp.float32),
            pltpu.VMEM((CH, D), jnp.float32),
            pltpu.VMEM((bpw, D), jnp.float32),
            pltpu.SemaphoreType.DMA,
            pltpu.SemaphoreType.DMA,
        ],
    )
    def k(idx_hbm, table_hbm, out_hbm, idx_v, buf0, buf1, acc_v, sem0, sem1):
        wid = lax.axis_index("s") * ncores + lax.axis_index("c")
        irow = wid * (NCH * bpw)
        pltpu.sync_copy(idx_hbm.at[pl.ds(irow, NCH * bpw)], idx_v)

        def reduce_chunk(buf, accs):
            unroll = 4
            def body(t, accs):
                accs = list(accs)
                for u in range(unroll):
                    s = t * unroll + u
                    for c in range(DV):
                        accs[c] = accs[c] + buf[s, pl.ds(c * LANES, LANES)]
                return tuple(accs)
            return lax.fori_loop(0, CH // unroll, body, tuple(accs))

        def row(i, _):
            cp0 = pltpu.async_copy(
                table_hbm.at[idx_v.at[NCH * i]], buf0, sem0)
            cp1 = pltpu.async_copy(
                table_hbm.at[idx_v.at[NCH * i + 1]], buf1, sem1)
            zeros = tuple(jnp.zeros((LANES,), jnp.float32) for _ in range(DV))
            cp0.wait()
            accs = reduce_chunk(buf0, zeros)
            cp1.wait()
            accs = reduce_chunk(buf1, accs)
            for c in range(DV):
                acc_v[i, pl.ds(c * LANES, LANES)] = accs[c]
            return 0

        lax.fori_loop(0, bpw, row, 0)
        pltpu.sync_copy(acc_v, out_hbm.at[pl.ds(wid * bpw, bpw)])

    return k(idx2d, table)


def _tail(pooled_sum, W, b2d):
    def body(ps_ref, w_ref, b_ref, o_ref):
        x = ps_ref[...] * (1.0 / S)
        logits = jnp.dot(x, w_ref[...], preferred_element_type=jnp.float32)
        logits = logits + b_ref[...]
        m = jnp.max(logits, axis=1, keepdims=True)
        e = jnp.exp(logits - m)
        lse = jnp.log(jnp.sum(e, axis=1, keepdims=True)) + m
        o_ref[...] = logits - lse

    return pl.pallas_call(
        body,
        out_shape=jax.ShapeDtypeStruct((B, NCLS), jnp.float32),
    )(pooled_sum, W, b2d)


def kernel(input, table, W, b):
    idx2d = input.astype(jnp.int32).reshape(B * NCH, CH)
    pooled_sum = _pooled_sum(idx2d, table)
    return _tail(pooled_sum, W, b.reshape(1, NCLS))


# speedup vs baseline: 1.1722x; 1.1722x over previous
import functools

import jax
import jax.numpy as jnp
from jax import lax
from jax.experimental import pallas as pl
from jax.experimental.pallas import tpu as pltpu
from jax.experimental.pallas import tpu_sc as plsc

B = 4096
S = 200
D = 64
NCLS = 10
CH = 100
NCH = S // CH
LANES = 16
DV = D // LANES


def _sc_info():
    try:
        info = plsc.get_sparse_core_info()
        return info.num_cores, info.num_subcores
    except Exception:
        return 2, 16


def _pooled_sum(idx2d, table):
    ncores, nsub = _sc_info()
    nw = ncores * nsub
    bpw = B // nw
    mesh = plsc.VectorSubcoreMesh(
        core_axis_name="c", subcore_axis_name="s",
        num_cores=ncores, num_subcores=nsub)

    nbuf = 8
    nchunks = NCH * bpw
    rows_per_g = nbuf // NCH

    @functools.partial(
        pl.kernel,
        out_type=jax.ShapeDtypeStruct((B, D), jnp.float32),
        mesh=mesh,
        compiler_params=pltpu.CompilerParams(use_tc_tiling_on_sc=False),
        scratch_types=[
            pltpu.VMEM((NCH * bpw, CH), jnp.int32),
            [pltpu.VMEM((CH, D), jnp.float32) for _ in range(nbuf)],
            pltpu.VMEM((bpw, D), jnp.float32),
            [pltpu.SemaphoreType.DMA for _ in range(nbuf)],
        ],
    )
    def k(idx_hbm, table_hbm, out_hbm, idx_v, bufs, acc_v, sems):
        wid = lax.axis_index("s") * ncores + lax.axis_index("c")
        irow = wid * nchunks
        pltpu.sync_copy(idx_hbm.at[pl.ds(irow, nchunks)], idx_v)

        def fire(j, slot):
            pltpu.async_copy(table_hbm.at[idx_v.at[j]], bufs[slot], sems[slot])

        def drain(j, slot):
            pltpu.make_async_copy(
                table_hbm.at[idx_v.at[j]], bufs[slot], sems[slot]).wait()

        def reduce_chunk(buf, accs):
            unroll = 4
            def body(t, accs):
                accs = list(accs)
                for u in range(unroll):
                    s = t * unroll + u
                    for c in range(DV):
                        accs[c] = accs[c] + buf[s, pl.ds(c * LANES, LANES)]
                return tuple(accs)
            return lax.fori_loop(0, CH // unroll, body, tuple(accs))

        for slot in range(nbuf):
            fire(slot, slot)

        def step(g, _):
            j0 = g * nbuf
            for r in range(rows_per_g):
                accs = tuple(jnp.zeros((LANES,), jnp.float32)
                             for _ in range(DV))
                for h in range(NCH):
                    slot = r * NCH + h
                    j = j0 + slot
                    drain(j, slot)
                    accs = reduce_chunk(bufs[slot], accs)

                    @pl.when(j + nbuf < nchunks)
                    def _():
                        fire(j + nbuf, slot)
                i = g * rows_per_g + r
                for c in range(DV):
                    acc_v[i, pl.ds(c * LANES, LANES)] = accs[c]
            return 0

        lax.fori_loop(0, nchunks // nbuf, step, 0)
        pltpu.sync_copy(acc_v, out_hbm.at[pl.ds(wid * bpw, bpw)])

    return k(idx2d, table)


def _tail(pooled_sum, W, b2d):
    def body(ps_ref, w_ref, b_ref, o_ref):
        x = ps_ref[...] * (1.0 / S)
        logits = jnp.dot(x, w_ref[...], preferred_element_type=jnp.float32)
        logits = logits + b_ref[...]
        m = jnp.max(logits, axis=1, keepdims=True)
        e = jnp.exp(logits - m)
        lse = jnp.log(jnp.sum(e, axis=1, keepdims=True)) + m
        o_ref[...] = logits - lse

    return pl.pallas_call(
        body,
        out_shape=jax.ShapeDtypeStruct((B, NCLS), jnp.float32),
    )(pooled_sum, W, b2d)


def kernel(input, table, W, b):
    idx2d = input.astype(jnp.int32).reshape(B * NCH, CH)
    pooled_sum = _pooled_sum(idx2d, table)
    return _tail(pooled_sum, W, b.reshape(1, NCLS))
